# baseline (speedup 1.0000x reference)
import functools

import jax
import jax.numpy as jnp
from jax.experimental import pallas as pl
from jax.experimental.pallas import tpu as pltpu

_LN_EPS = 1e-5
_POOL_EPS = 1e-6


def _mish(x):
    u = jnp.exp(jnp.minimum(x, 20.0))
    t = u * (u + 2.0)
    return x * (t / (t + 2.0))


def _layer_norm(x, g, b):
    m = jnp.mean(x, axis=-1, keepdims=True)
    msq = jnp.mean(x * x, axis=-1, keepdims=True)
    v = jnp.maximum(msq - m * m, 0.0)
    k = jax.lax.rsqrt(v + _LN_EPS) * g
    return x * k - m * k + b


def _fused_kernel(starts_ref, ends_ref, x_ref, w1_ref, b1_ref, lng_ref,
                  lnb_ref, w2_ref, b2_ref, g1_ref, gb1_ref, ln2g_ref,
                  ln2b_ref, g2_ref, gb2_ref, g3_ref, gb3_ref, out_ref,
                  acc_ref, *, num_images):
    s = pl.program_id(0)
    n_s = pl.num_programs(0)
    n_b = x_ref.shape[0]
    ts = x_ref.shape[1]
    a = w1_ref.shape[1]

    for b in range(n_b):
        xc = x_ref[b].astype(jnp.bfloat16)
        h = jnp.dot(xc, w1_ref[...], preferred_element_type=jnp.float32)
        h = h + b1_ref[...]
        h = _mish(_layer_norm(h, lng_ref[...], lnb_ref[...]))
        adapted = jnp.dot(h.astype(jnp.bfloat16), w2_ref[...],
                          preferred_element_type=jnp.float32)
        adapted = adapted + b2_ref[...]
        geo = adapted[:, :a].astype(jnp.bfloat16)
        wrow = jnp.transpose(adapted[:, a:a + 1], (1, 0))
        wv = jax.nn.sigmoid(wrow)

        pos = s * ts + jax.lax.broadcasted_iota(jnp.int32, (8, ts), 1)
        row = jax.lax.broadcasted_iota(jnp.int32, (8, ts), 0)
        mask = jnp.zeros((8, ts), jnp.float32)
        for i in range(num_images):
            hit = ((pos >= starts_ref[b, i]) & (pos < ends_ref[b, i])
                   & (row == i))
            mask = mask + hit.astype(jnp.float32)

        wmask = (mask * wv).astype(jnp.bfloat16)
        part_num = jnp.dot(wmask, geo, preferred_element_type=jnp.float32)
        part_den = jnp.sum(wmask.astype(jnp.float32), axis=1, keepdims=True)

        @pl.when(s == 0)
        def _():
            acc_ref[b * 8:(b + 1) * 8, :a] = part_num
            acc_ref[b * 8:(b + 1) * 8, a:a + 1] = part_den

        @pl.when(s != 0)
        def _():
            acc_ref[b * 8:(b + 1) * 8, :a] += part_num
            acc_ref[b * 8:(b + 1) * 8, a:a + 1] += part_den

    @pl.when(s == n_s - 1)
    def _():
        pr = acc_ref[...]
        num = pr[:, :a]
        den = pr[:, a:a + 1]
        pooled = num / (den + _POOL_EPS)
        g = jnp.dot(pooled, g1_ref[...], preferred_element_type=jnp.float32)
        g = _mish(_layer_norm(g + gb1_ref[...], ln2g_ref[...], ln2b_ref[...]))
        g = _mish(jnp.dot(g, g2_ref[...], preferred_element_type=jnp.float32)
                  + gb2_ref[...])
        raw = jnp.dot(g, g3_ref[...], preferred_element_type=jnp.float32)
        raw = raw + gb3_ref[...]
        lane = jax.lax.broadcasted_iota(jnp.int32, raw.shape, 1)
        qsq = jnp.where((lane >= 3) & (lane < 7), raw * raw, 0.0)
        nrm = jnp.sqrt(jnp.sum(qsq, axis=-1, keepdims=True))
        scale = jnp.where(lane >= 3, 1.0 / jnp.maximum(nrm, 1e-12), 1.0)
        out_ref[...] = raw * scale


def kernel(hidden_states, batch_image_tuples, params):
    B, S, H = hidden_states.shape
    nI = batch_image_tuples.shape[1]
    A = params["W1"].shape[1]
    O = params["G1"].shape[1]
    Oh = params["G2"].shape[1]
    num_images = B * nI
    if num_images == 0:
        return (jnp.zeros((0, 3), jnp.float32), jnp.zeros((0, 4), jnp.float32))

    TS = 128 if S % 128 == 0 else S
    n_s = S // TS
    R = B * 8

    starts = batch_image_tuples[..., 0].astype(jnp.int32)
    ends = batch_image_tuples[..., 1].astype(jnp.int32)

    W2 = params["W2"]
    W1b = params["W1"].astype(jnp.bfloat16)
    W2p = jnp.concatenate(
        [W2[:, :A], W2[:, A:A + 1],
         jnp.zeros((A, 127), jnp.float32)], axis=1).astype(jnp.bfloat16)
    b2p = jnp.concatenate(
        [params["b2"], jnp.zeros((127,), jnp.float32)])[None, :]
    G3p = jnp.concatenate(
        [params["G3"], jnp.zeros((Oh, 121), jnp.float32)], axis=1)
    gb3p = jnp.concatenate(
        [params["gb3"], jnp.zeros((121,), jnp.float32)])[None, :]

    smem = pl.BlockSpec(memory_space=pltpu.SMEM)
    const2 = lambda shape: pl.BlockSpec(shape, lambda s: (0, 0))

    out = pl.pallas_call(
        functools.partial(_fused_kernel, num_images=nI),
        grid=(n_s,),
        in_specs=[
            smem,
            smem,
            pl.BlockSpec((B, TS, H), lambda s: (0, s, 0)),
            const2((H, A)),
            const2((1, A)),
            const2((1, A)),
            const2((1, A)),
            const2((A, A + 128)),
            const2((1, A + 128)),
            const2((A, O)),
            const2((1, O)),
            const2((1, O)),
            const2((1, O)),
            const2((O, Oh)),
            const2((1, Oh)),
            const2((Oh, 128)),
            const2((1, 128)),
        ],
        out_specs=pl.BlockSpec((R, 128), lambda s: (0, 0)),
        out_shape=jax.ShapeDtypeStruct((R, 128), jnp.float32),
        scratch_shapes=[pltpu.VMEM((R, A + 128), jnp.float32)],
    )(
        starts, ends, hidden_states, W1b,
        params["b1"][None, :], params["ln1_g"][None, :],
        params["ln1_b"][None, :], W2p, b2p,
        params["G1"], params["gb1"][None, :], params["ln2_g"][None, :],
        params["ln2_b"][None, :], params["G2"], params["gb2"][None, :],
        G3p, gb3p,
    )

    res = out.reshape(B, 8, 128)[:, :nI].reshape(num_images, 128)
    return (res[:, :3], res[:, 3:7])


# speedup vs baseline: 1.2985x; 1.2985x over previous
import functools

import jax
import jax.numpy as jnp
from jax.experimental import pallas as pl
from jax.experimental.pallas import tpu as pltpu

_LN_EPS = 1e-5
_POOL_EPS = 1e-6


def _mish(x):
    u = jnp.exp(jnp.minimum(x, 20.0))
    t = u * (u + 2.0)
    return x * (t / (t + 2.0))


def _layer_norm(x, g, b):
    m = jnp.mean(x, axis=-1, keepdims=True)
    msq = jnp.mean(x * x, axis=-1, keepdims=True)
    v = jnp.maximum(msq - m * m, 0.0)
    k = jax.lax.rsqrt(v + _LN_EPS) * g
    return x * k - m * k + b


def _fused_kernel(starts_ref, ends_ref, x_ref, w1_ref, b1_ref, lng_ref,
                  lnb_ref, w2_ref, b2_ref, g1_ref, gb1_ref, ln2g_ref,
                  ln2b_ref, g2_ref, gb2_ref, g3_ref, gb3_ref, out_ref,
                  acc_ref, *, num_images):
    s = pl.program_id(0)
    n_s = pl.num_programs(0)
    n_b = x_ref.shape[0]
    ts = x_ref.shape[1]
    a = w1_ref.shape[1]

    for b in range(n_b):
        xc = x_ref[b].astype(jnp.bfloat16)
        h = jnp.dot(xc, w1_ref[...], preferred_element_type=jnp.float32)
        h = h + b1_ref[...]
        m = jnp.mean(h, axis=-1, keepdims=True)
        msq = jnp.mean(h * h, axis=-1, keepdims=True)
        v = jnp.maximum(msq - m * m, 0.0)
        k = jax.lax.rsqrt(v + _LN_EPS)
        hb = ((h - m) * k).astype(jnp.bfloat16)
        hb = hb * lng_ref[...] + lnb_ref[...]
        hb = _mish(hb)
        adapted = jnp.dot(hb, w2_ref[...],
                          preferred_element_type=jnp.float32)
        adapted = adapted + b2_ref[...]
        geo = adapted[:, :a].astype(jnp.bfloat16)
        wrow = jnp.transpose(adapted[:, a:a + 1], (1, 0))
        wv = jax.nn.sigmoid(wrow)

        pos = s * ts + jax.lax.broadcasted_iota(jnp.int32, (8, ts), 1)
        row = jax.lax.broadcasted_iota(jnp.int32, (8, ts), 0)
        mask = jnp.zeros((8, ts), jnp.float32)
        for i in range(num_images):
            hit = ((pos >= starts_ref[b, i]) & (pos < ends_ref[b, i])
                   & (row == i))
            mask = mask + hit.astype(jnp.float32)

        wmask = (mask * wv).astype(jnp.bfloat16)
        part_num = jnp.dot(wmask, geo, preferred_element_type=jnp.float32)
        part_den = jnp.sum(wmask.astype(jnp.float32), axis=1, keepdims=True)

        @pl.when(s == 0)
        def _():
            acc_ref[b * 8:(b + 1) * 8, :a] = part_num
            acc_ref[b * 8:(b + 1) * 8, a:a + 1] = part_den

        @pl.when(s != 0)
        def _():
            acc_ref[b * 8:(b + 1) * 8, :a] += part_num
            acc_ref[b * 8:(b + 1) * 8, a:a + 1] += part_den

    @pl.when(s == n_s - 1)
    def _():
        pr = acc_ref[...]
        num = pr[:, :a]
        den = pr[:, a:a + 1]
        pooled = num / (den + _POOL_EPS)
        g = jnp.dot(pooled, g1_ref[...], preferred_element_type=jnp.float32)
        g = _mish(_layer_norm(g + gb1_ref[...], ln2g_ref[...], ln2b_ref[...]))
        g = _mish(jnp.dot(g, g2_ref[...], preferred_element_type=jnp.float32)
                  + gb2_ref[...])
        raw = jnp.dot(g, g3_ref[...], preferred_element_type=jnp.float32)
        raw = raw + gb3_ref[...]
        lane = jax.lax.broadcasted_iota(jnp.int32, raw.shape, 1)
        qsq = jnp.where((lane >= 3) & (lane < 7), raw * raw, 0.0)
        nrm = jnp.sqrt(jnp.sum(qsq, axis=-1, keepdims=True))
        scale = jnp.where(lane >= 3, 1.0 / jnp.maximum(nrm, 1e-12), 1.0)
        out_ref[...] = raw * scale


def kernel(hidden_states, batch_image_tuples, params):
    B, S, H = hidden_states.shape
    nI = batch_image_tuples.shape[1]
    A = params["W1"].shape[1]
    O = params["G1"].shape[1]
    Oh = params["G2"].shape[1]
    num_images = B * nI
    if num_images == 0:
        return (jnp.zeros((0, 3), jnp.float32), jnp.zeros((0, 4), jnp.float32))

    TS = 256 if S % 256 == 0 else S
    n_s = S // TS
    R = B * 8

    starts = batch_image_tuples[..., 0].astype(jnp.int32)
    ends = batch_image_tuples[..., 1].astype(jnp.int32)

    W2 = params["W2"]
    W1b = params["W1"].astype(jnp.bfloat16)
    W2p = jnp.concatenate(
        [W2[:, :A], W2[:, A:A + 1],
         jnp.zeros((A, 127), jnp.float32)], axis=1).astype(jnp.bfloat16)
    b2p = jnp.concatenate(
        [params["b2"], jnp.zeros((127,), jnp.float32)])[None, :]
    G3p = jnp.concatenate(
        [params["G3"], jnp.zeros((Oh, 121), jnp.float32)], axis=1)
    gb3p = jnp.concatenate(
        [params["gb3"], jnp.zeros((121,), jnp.float32)])[None, :]

    smem = pl.BlockSpec(memory_space=pltpu.SMEM)
    const2 = lambda shape: pl.BlockSpec(shape, lambda s: (0, 0))

    out = pl.pallas_call(
        functools.partial(_fused_kernel, num_images=nI),
        grid=(n_s,),
        in_specs=[
            smem,
            smem,
            pl.BlockSpec((B, TS, H), lambda s: (0, s, 0)),
            const2((H, A)),
            const2((1, A)),
            const2((1, A)),
            const2((1, A)),
            const2((A, A + 128)),
            const2((1, A + 128)),
            const2((A, O)),
            const2((1, O)),
            const2((1, O)),
            const2((1, O)),
            const2((O, Oh)),
            const2((1, Oh)),
            const2((Oh, 128)),
            const2((1, 128)),
        ],
        out_specs=pl.BlockSpec((R, 128), lambda s: (0, 0)),
        out_shape=jax.ShapeDtypeStruct((R, 128), jnp.float32),
        scratch_shapes=[pltpu.VMEM((R, A + 128), jnp.float32)],
    )(
        starts, ends, hidden_states, W1b,
        params["b1"][None, :],
        params["ln1_g"][None, :].astype(jnp.bfloat16),
        params["ln1_b"][None, :].astype(jnp.bfloat16), W2p, b2p,
        params["G1"], params["gb1"][None, :], params["ln2_g"][None, :],
        params["ln2_b"][None, :], params["G2"], params["gb2"][None, :],
        G3p, gb3p,
    )

    res = out.reshape(B, 8, 128)[:, :nI].reshape(num_images, 128)
    return (res[:, :3], res[:, 3:7])
